# Initial kernel scaffold; baseline (speedup 1.0000x reference)
import jax
import jax.numpy as jnp
from jax.experimental import pallas as pl


def kernel(emb, edge_index, d, scale):
    raise NotImplementedError("write your pallas kernel here")


# speedup vs baseline: 2.8847x; 2.8847x over previous
import functools

import jax
import jax.numpy as jnp
from jax import lax
from jax.experimental import pallas as pl
from jax.experimental.pallas import tpu as pltpu
from jax.experimental.pallas import tpu_sc as plsc

N_NODES = 10000
N_HID = 128
N_EDGES = 320000
NW = 32
PER_W = N_EDGES // NW
CHUNK = 80
NCHUNK = PER_W // CHUNK
LANES = 16
NVEC = N_HID // LANES


def _prep_body(emb_ref, ds_ref, a_ref, b_ref):
    x = emb_ref[...]
    n2 = jnp.sum(x * x, axis=1, keepdims=True)
    inv = 1.0 / jnp.maximum(jnp.sqrt(n2), 1e-12)
    xn = x * inv
    b_ref[...] = xn
    a_ref[...] = xn * ds_ref[...]


_ROWS_PER_BLK = 1000


def _prep(emb, ds):
    return pl.pallas_call(
        _prep_body,
        grid=(N_NODES // _ROWS_PER_BLK,),
        in_specs=[
            pl.BlockSpec((_ROWS_PER_BLK, N_HID), lambda i: (i, 0)),
            pl.BlockSpec((1, N_HID), lambda i: (0, 0)),
        ],
        out_specs=[
            pl.BlockSpec((_ROWS_PER_BLK, N_HID), lambda i: (i, 0)),
            pl.BlockSpec((_ROWS_PER_BLK, N_HID), lambda i: (i, 0)),
        ],
        out_shape=[
            jax.ShapeDtypeStruct((N_NODES, N_HID), jnp.float32),
            jax.ShapeDtypeStruct((N_NODES, N_HID), jnp.float32),
        ],
    )(emb, ds)


def _sc_body(a_hbm, b_hbm, src_hbm, dst_hbm, out_hbm,
             src_v, dst_v, arows_v, brows_v, out_v, sem_a, sem_b):
    cid = lax.axis_index("c")
    sid = lax.axis_index("s")
    wid = sid * 2 + cid
    base = wid * PER_W

    pltpu.sync_copy(src_hbm.at[pl.ds(base, PER_W)], src_v)
    pltpu.sync_copy(dst_hbm.at[pl.ds(base, PER_W)], dst_v)

    iota16 = lax.iota(jnp.int32, LANES)

    def chunk_body(c, carry):
        off = c * CHUNK
        cp_a = pltpu.async_copy(
            a_hbm.at[src_v.at[pl.ds(off, CHUNK)]], arows_v, sem_a)
        cp_b = pltpu.async_copy(
            b_hbm.at[dst_v.at[pl.ds(off, CHUNK)]], brows_v, sem_b)
        cp_a.wait()
        cp_b.wait()

        def group_body(g, carry2):
            eoff = g * LANES
            res = jnp.zeros((LANES,), jnp.float32)
            for es in range(LANES):
                e = eoff + es
                acc = (arows_v[e, pl.ds(0, LANES)] *
                       brows_v[e, pl.ds(0, LANES)])
                for j in range(1, NVEC):
                    acc = acc + (arows_v[e, pl.ds(j * LANES, LANES)] *
                                 brows_v[e, pl.ds(j * LANES, LANES)])
                res = jnp.where(iota16 == es, jnp.sum(acc), res)
            out_v[pl.ds(off + eoff, LANES)] = res
            return carry2

        lax.fori_loop(0, CHUNK // LANES, group_body, 0)
        return carry

    lax.fori_loop(0, NCHUNK, chunk_body, 0)
    pltpu.sync_copy(out_v, out_hbm.at[pl.ds(base, PER_W)])


@functools.partial(
    pl.kernel,
    out_type=jax.ShapeDtypeStruct((N_EDGES,), jnp.float32),
    mesh=plsc.VectorSubcoreMesh(core_axis_name="c", subcore_axis_name="s"),
    compiler_params=pltpu.CompilerParams(needs_layout_passes=False),
    scratch_types=[
        pltpu.VMEM((PER_W,), jnp.int32),
        pltpu.VMEM((PER_W,), jnp.int32),
        pltpu.VMEM((CHUNK, N_HID), jnp.float32),
        pltpu.VMEM((CHUNK, N_HID), jnp.float32),
        pltpu.VMEM((PER_W,), jnp.float32),
        pltpu.SemaphoreType.DMA,
        pltpu.SemaphoreType.DMA,
    ],
)
def _sc_call(a_hbm, b_hbm, src_hbm, dst_hbm, out_hbm,
             src_v, dst_v, arows_v, brows_v, out_v, sem_a, sem_b):
    _sc_body(a_hbm, b_hbm, src_hbm, dst_hbm, out_hbm,
             src_v, dst_v, arows_v, brows_v, out_v, sem_a, sem_b)


def kernel(emb, edge_index, d, scale):
    ds = (d * scale).reshape(1, N_HID)
    a, b = _prep(emb, ds)
    src = edge_index[0].astype(jnp.int32)
    dst = edge_index[1].astype(jnp.int32)
    z = _sc_call(a, b, src, dst)
    return z.reshape(N_EDGES, 1)


# speedup vs baseline: 4.2366x; 1.4687x over previous
import functools

import jax
import jax.numpy as jnp
from jax import lax
from jax.experimental import pallas as pl
from jax.experimental.pallas import tpu as pltpu
from jax.experimental.pallas import tpu_sc as plsc

N_NODES = 10000
N_HID = 128
N_EDGES = 320000
NW = 32
PER_W = N_EDGES // NW
CHUNK = 80
NCHUNK = PER_W // CHUNK
LANES = 16
NVEC = N_HID // LANES


def _prep_body(emb_ref, ds_ref, a_ref, b_ref):
    x = emb_ref[...]
    n2 = jnp.sum(x * x, axis=1, keepdims=True)
    inv = 1.0 / jnp.maximum(jnp.sqrt(n2), 1e-12)
    xn = x * inv
    b_ref[...] = xn
    a_ref[...] = xn * ds_ref[...]


_ROWS_PER_BLK = 1000


def _prep(emb, ds):
    return pl.pallas_call(
        _prep_body,
        grid=(N_NODES // _ROWS_PER_BLK,),
        in_specs=[
            pl.BlockSpec((_ROWS_PER_BLK, N_HID), lambda i: (i, 0)),
            pl.BlockSpec((1, N_HID), lambda i: (0, 0)),
        ],
        out_specs=[
            pl.BlockSpec((_ROWS_PER_BLK, N_HID), lambda i: (i, 0)),
            pl.BlockSpec((_ROWS_PER_BLK, N_HID), lambda i: (i, 0)),
        ],
        out_shape=[
            jax.ShapeDtypeStruct((N_NODES, N_HID), jnp.float32),
            jax.ShapeDtypeStruct((N_NODES, N_HID), jnp.float32),
        ],
    )(emb, ds)


def _sc_body(a_hbm, b_hbm, src_hbm, dst_hbm, out_hbm,
             src_v, dst_v, arows_v, brows_v, out_v, sem_a0, sem_b0,
             sem_a1, sem_b1):
    cid = lax.axis_index("c")
    sid = lax.axis_index("s")
    wid = sid * 2 + cid
    base = wid * PER_W

    pltpu.sync_copy(src_hbm.at[pl.ds(base, PER_W)], src_v)
    pltpu.sync_copy(dst_hbm.at[pl.ds(base, PER_W)], dst_v)

    iota16 = lax.iota(jnp.int32, LANES)
    sems = ((sem_a0, sem_b0), (sem_a1, sem_b1))

    def start(c, b):
        off = c * CHUNK
        pltpu.async_copy(
            a_hbm.at[src_v.at[pl.ds(off, CHUNK)]],
            arows_v.at[pl.ds(b * CHUNK, CHUNK)], sems[b][0])
        pltpu.async_copy(
            b_hbm.at[dst_v.at[pl.ds(off, CHUNK)]],
            brows_v.at[pl.ds(b * CHUNK, CHUNK)], sems[b][1])

    def wait(b):
        pltpu.make_async_copy(
            a_hbm.at[src_v.at[pl.ds(0, CHUNK)]],
            arows_v.at[pl.ds(b * CHUNK, CHUNK)], sems[b][0]).wait()
        pltpu.make_async_copy(
            b_hbm.at[dst_v.at[pl.ds(0, CHUNK)]],
            brows_v.at[pl.ds(b * CHUNK, CHUNK)], sems[b][1]).wait()

    start(0, 0)

    def chunk_body(c, carry):
        off = c * CHUNK
        parity = lax.rem(c, 2)

        @pl.when(c + 1 < NCHUNK)
        def _():
            @pl.when(parity == 0)
            def _():
                start(c + 1, 1)

            @pl.when(parity == 1)
            def _():
                start(c + 1, 0)

        @pl.when(parity == 0)
        def _():
            wait(0)

        @pl.when(parity == 1)
        def _():
            wait(1)

        bofs = parity * CHUNK

        def group_body(g, carry2):
            eoff = g * LANES
            res = jnp.zeros((LANES,), jnp.float32)
            for es in range(LANES):
                e = bofs + eoff + es
                acc = (arows_v[e, pl.ds(0, LANES)] *
                       brows_v[e, pl.ds(0, LANES)])
                for j in range(1, NVEC):
                    acc = acc + (arows_v[e, pl.ds(j * LANES, LANES)] *
                                 brows_v[e, pl.ds(j * LANES, LANES)])
                res = jnp.where(iota16 == es, jnp.sum(acc), res)
            out_v[pl.ds(off + eoff, LANES)] = res
            return carry2

        lax.fori_loop(0, CHUNK // LANES, group_body, 0)
        return carry

    lax.fori_loop(0, NCHUNK, chunk_body, 0)
    pltpu.sync_copy(out_v, out_hbm.at[pl.ds(base, PER_W)])


@functools.partial(
    pl.kernel,
    out_type=jax.ShapeDtypeStruct((N_EDGES,), jnp.float32),
    mesh=plsc.VectorSubcoreMesh(core_axis_name="c", subcore_axis_name="s"),
    compiler_params=pltpu.CompilerParams(needs_layout_passes=False),
    scratch_types=[
        pltpu.VMEM((PER_W,), jnp.int32),
        pltpu.VMEM((PER_W,), jnp.int32),
        pltpu.VMEM((2 * CHUNK, N_HID), jnp.float32),
        pltpu.VMEM((2 * CHUNK, N_HID), jnp.float32),
        pltpu.VMEM((PER_W,), jnp.float32),
        pltpu.SemaphoreType.DMA,
        pltpu.SemaphoreType.DMA,
        pltpu.SemaphoreType.DMA,
        pltpu.SemaphoreType.DMA,
    ],
)
def _sc_call(a_hbm, b_hbm, src_hbm, dst_hbm, out_hbm,
             src_v, dst_v, arows_v, brows_v, out_v, sem_a0, sem_b0,
             sem_a1, sem_b1):
    _sc_body(a_hbm, b_hbm, src_hbm, dst_hbm, out_hbm,
             src_v, dst_v, arows_v, brows_v, out_v, sem_a0, sem_b0,
             sem_a1, sem_b1)


def kernel(emb, edge_index, d, scale):
    ds = (d * scale).reshape(1, N_HID)
    a, b = _prep(emb, ds)
    src = edge_index[0].astype(jnp.int32)
    dst = edge_index[1].astype(jnp.int32)
    z = _sc_call(a, b, src, dst)
    return z.reshape(N_EDGES, 1)


# speedup vs baseline: 6.7708x; 1.5982x over previous
import functools

import jax
import jax.numpy as jnp
from jax import lax
from jax.experimental import pallas as pl
from jax.experimental.pallas import tpu as pltpu
from jax.experimental.pallas import tpu_sc as plsc

N_NODES = 10000
N_HID = 128
N_EDGES = 320000
NW = 32
PER_W = N_EDGES // NW
CHUNK = 80
NCHUNK = PER_W // CHUNK
LANES = 16
NVEC = N_HID // LANES


def _prep_body(emb_ref, ds_ref, a_ref, b_ref):
    x = emb_ref[...]
    n2 = jnp.sum(x * x, axis=1, keepdims=True)
    inv = 1.0 / jnp.maximum(jnp.sqrt(n2), 1e-12)
    xn = x * inv
    b_ref[...] = xn
    a_ref[...] = xn * ds_ref[...]


_ROWS_PER_BLK = 1000


def _prep(emb, ds):
    return pl.pallas_call(
        _prep_body,
        grid=(N_NODES // _ROWS_PER_BLK,),
        in_specs=[
            pl.BlockSpec((_ROWS_PER_BLK, N_HID), lambda i: (i, 0)),
            pl.BlockSpec((1, N_HID), lambda i: (0, 0)),
        ],
        out_specs=[
            pl.BlockSpec((_ROWS_PER_BLK, N_HID), lambda i: (i, 0)),
            pl.BlockSpec((_ROWS_PER_BLK, N_HID), lambda i: (i, 0)),
        ],
        out_shape=[
            jax.ShapeDtypeStruct((N_NODES, N_HID), jnp.float32),
            jax.ShapeDtypeStruct((N_NODES, N_HID), jnp.float32),
        ],
    )(emb, ds)


def _sc_body(a_hbm, b_hbm, src_hbm, dst_hbm, out_hbm,
             src_v, dst_v, arows_v, brows_v, tile_v, out_v, sem_a0, sem_b0,
             sem_a1, sem_b1):
    cid = lax.axis_index("c")
    sid = lax.axis_index("s")
    wid = sid * 2 + cid
    base = wid * PER_W

    pltpu.sync_copy(src_hbm.at[pl.ds(base, PER_W)], src_v)
    pltpu.sync_copy(dst_hbm.at[pl.ds(base, PER_W)], dst_v)

    iota16 = lax.iota(jnp.int32, LANES)
    sems = ((sem_a0, sem_b0), (sem_a1, sem_b1))

    def start(c, b):
        off = c * CHUNK
        pltpu.async_copy(
            a_hbm.at[src_v.at[pl.ds(off, CHUNK)]],
            arows_v.at[pl.ds(b * CHUNK, CHUNK)], sems[b][0])
        pltpu.async_copy(
            b_hbm.at[dst_v.at[pl.ds(off, CHUNK)]],
            brows_v.at[pl.ds(b * CHUNK, CHUNK)], sems[b][1])

    def wait(b):
        pltpu.make_async_copy(
            a_hbm.at[src_v.at[pl.ds(0, CHUNK)]],
            arows_v.at[pl.ds(b * CHUNK, CHUNK)], sems[b][0]).wait()
        pltpu.make_async_copy(
            b_hbm.at[dst_v.at[pl.ds(0, CHUNK)]],
            brows_v.at[pl.ds(b * CHUNK, CHUNK)], sems[b][1]).wait()

    start(0, 0)

    def chunk_body(c, carry):
        off = c * CHUNK
        parity = lax.rem(c, 2)

        @pl.when(c + 1 < NCHUNK)
        def _():
            @pl.when(parity == 0)
            def _():
                start(c + 1, 1)

            @pl.when(parity == 1)
            def _():
                start(c + 1, 0)

        @pl.when(parity == 0)
        def _():
            wait(0)

        @pl.when(parity == 1)
        def _():
            wait(1)

        bofs = parity * CHUNK

        def group_body(g, carry2):
            eoff = g * LANES
            for es in range(LANES):
                e = bofs + eoff + es
                prods = [arows_v[e, pl.ds(j * LANES, LANES)] *
                         brows_v[e, pl.ds(j * LANES, LANES)]
                         for j in range(NVEC)]
                while len(prods) > 1:
                    prods = [prods[i] + prods[i + 1]
                             for i in range(0, len(prods), 2)]
                plsc.store_scatter(tile_v, [iota16 * LANES + es], prods[0])
            cols = [tile_v[pl.ds(c2 * LANES, LANES)] for c2 in range(LANES)]
            while len(cols) > 1:
                cols = [cols[i] + cols[i + 1] for i in range(0, len(cols), 2)]
            out_v[pl.ds(off + eoff, LANES)] = cols[0]
            return carry2

        lax.fori_loop(0, CHUNK // LANES, group_body, 0)
        return carry

    lax.fori_loop(0, NCHUNK, chunk_body, 0)
    pltpu.sync_copy(out_v, out_hbm.at[pl.ds(base, PER_W)])


@functools.partial(
    pl.kernel,
    out_type=jax.ShapeDtypeStruct((N_EDGES,), jnp.float32),
    mesh=plsc.VectorSubcoreMesh(core_axis_name="c", subcore_axis_name="s"),
    compiler_params=pltpu.CompilerParams(needs_layout_passes=False),
    scratch_types=[
        pltpu.VMEM((PER_W,), jnp.int32),
        pltpu.VMEM((PER_W,), jnp.int32),
        pltpu.VMEM((2 * CHUNK, N_HID), jnp.float32),
        pltpu.VMEM((2 * CHUNK, N_HID), jnp.float32),
        pltpu.VMEM((LANES * LANES,), jnp.float32),
        pltpu.VMEM((PER_W,), jnp.float32),
        pltpu.SemaphoreType.DMA,
        pltpu.SemaphoreType.DMA,
        pltpu.SemaphoreType.DMA,
        pltpu.SemaphoreType.DMA,
    ],
)
def _sc_call(a_hbm, b_hbm, src_hbm, dst_hbm, out_hbm,
             src_v, dst_v, arows_v, brows_v, tile_v, out_v, sem_a0, sem_b0,
             sem_a1, sem_b1):
    _sc_body(a_hbm, b_hbm, src_hbm, dst_hbm, out_hbm,
             src_v, dst_v, arows_v, brows_v, tile_v, out_v, sem_a0, sem_b0,
             sem_a1, sem_b1)


def kernel(emb, edge_index, d, scale):
    ds = (d * scale).reshape(1, N_HID)
    a, b = _prep(emb, ds)
    src = edge_index[0].astype(jnp.int32)
    dst = edge_index[1].astype(jnp.int32)
    z = _sc_call(a, b, src, dst)
    return z.reshape(N_EDGES, 1)


# speedup vs baseline: 7.3908x; 1.0916x over previous
import functools

import jax
import jax.numpy as jnp
from jax import lax
from jax.experimental import pallas as pl
from jax.experimental.pallas import tpu as pltpu
from jax.experimental.pallas import tpu_sc as plsc

N_NODES = 10000
N_HID = 128
N_EDGES = 320000
NW = 32
PER_W = N_EDGES // NW
CHUNK = 80
NCHUNK = PER_W // CHUNK
LANES = 16
NVEC = N_HID // LANES


def _prep_body(emb_ref, ds_ref, a_ref, b_ref):
    x = emb_ref[...]
    n2 = jnp.sum(x * x, axis=1, keepdims=True)
    inv = 1.0 / jnp.maximum(jnp.sqrt(n2), 1e-12)
    xn = x * inv
    b_ref[...] = xn.astype(jnp.bfloat16)
    a_ref[...] = (xn * ds_ref[...]).astype(jnp.bfloat16)


_ROWS_PER_BLK = 1000


def _prep(emb, ds):
    return pl.pallas_call(
        _prep_body,
        grid=(N_NODES // _ROWS_PER_BLK,),
        in_specs=[
            pl.BlockSpec((_ROWS_PER_BLK, N_HID), lambda i: (i, 0)),
            pl.BlockSpec((1, N_HID), lambda i: (0, 0)),
        ],
        out_specs=[
            pl.BlockSpec((_ROWS_PER_BLK, N_HID), lambda i: (i, 0)),
            pl.BlockSpec((_ROWS_PER_BLK, N_HID), lambda i: (i, 0)),
        ],
        out_shape=[
            jax.ShapeDtypeStruct((N_NODES, N_HID), jnp.bfloat16),
            jax.ShapeDtypeStruct((N_NODES, N_HID), jnp.bfloat16),
        ],
    )(emb, ds)


def _sc_body(a_hbm, b_hbm, src_hbm, dst_hbm, out_hbm,
             src_v, dst_v, arows_v, brows_v, tile_v, out_v, sem_a0, sem_b0,
             sem_a1, sem_b1):
    cid = lax.axis_index("c")
    sid = lax.axis_index("s")
    wid = sid * 2 + cid
    base = wid * PER_W

    pltpu.sync_copy(src_hbm.at[pl.ds(base, PER_W)], src_v)
    pltpu.sync_copy(dst_hbm.at[pl.ds(base, PER_W)], dst_v)

    iota16 = lax.iota(jnp.int32, LANES)
    sems = ((sem_a0, sem_b0), (sem_a1, sem_b1))

    def start(c, b):
        off = c * CHUNK
        pltpu.async_copy(
            a_hbm.at[src_v.at[pl.ds(off, CHUNK)]],
            arows_v.at[pl.ds(b * CHUNK, CHUNK)], sems[b][0])
        pltpu.async_copy(
            b_hbm.at[dst_v.at[pl.ds(off, CHUNK)]],
            brows_v.at[pl.ds(b * CHUNK, CHUNK)], sems[b][1])

    def wait(b):
        pltpu.make_async_copy(
            a_hbm.at[src_v.at[pl.ds(0, CHUNK)]],
            arows_v.at[pl.ds(b * CHUNK, CHUNK)], sems[b][0]).wait()
        pltpu.make_async_copy(
            b_hbm.at[dst_v.at[pl.ds(0, CHUNK)]],
            brows_v.at[pl.ds(b * CHUNK, CHUNK)], sems[b][1]).wait()

    start(0, 0)

    def chunk_body(c, carry):
        off = c * CHUNK
        parity = lax.rem(c, 2)

        @pl.when(c + 1 < NCHUNK)
        def _():
            @pl.when(parity == 0)
            def _():
                start(c + 1, 1)

            @pl.when(parity == 1)
            def _():
                start(c + 1, 0)

        @pl.when(parity == 0)
        def _():
            wait(0)

        @pl.when(parity == 1)
        def _():
            wait(1)

        bofs = parity * CHUNK

        def group_body(g, carry2):
            eoff = g * LANES
            for es in range(LANES):
                e = bofs + eoff + es
                prods = []
                for j in range(NVEC // 2):
                    a_pk = arows_v[e, pl.ds(j * 2 * LANES, 2 * LANES)]
                    b_pk = brows_v[e, pl.ds(j * 2 * LANES, 2 * LANES)]
                    a0, a1 = plsc.unpack(
                        a_pk, format=plsc.PackFormat.INTERLEAVED)
                    b0, b1 = plsc.unpack(
                        b_pk, format=plsc.PackFormat.INTERLEAVED)
                    prods.append(a0 * b0)
                    prods.append(a1 * b1)
                while len(prods) > 1:
                    prods = [prods[i] + prods[i + 1]
                             for i in range(0, len(prods), 2)]
                plsc.store_scatter(tile_v, [iota16 * LANES + es], prods[0])
            cols = [tile_v[pl.ds(c2 * LANES, LANES)] for c2 in range(LANES)]
            while len(cols) > 1:
                cols = [cols[i] + cols[i + 1] for i in range(0, len(cols), 2)]
            out_v[pl.ds(off + eoff, LANES)] = cols[0]
            return carry2

        lax.fori_loop(0, CHUNK // LANES, group_body, 0)
        return carry

    lax.fori_loop(0, NCHUNK, chunk_body, 0)
    pltpu.sync_copy(out_v, out_hbm.at[pl.ds(base, PER_W)])


@functools.partial(
    pl.kernel,
    out_type=jax.ShapeDtypeStruct((N_EDGES,), jnp.float32),
    mesh=plsc.VectorSubcoreMesh(core_axis_name="c", subcore_axis_name="s"),
    compiler_params=pltpu.CompilerParams(
        needs_layout_passes=False, use_tc_tiling_on_sc=False),
    scratch_types=[
        pltpu.VMEM((PER_W,), jnp.int32),
        pltpu.VMEM((PER_W,), jnp.int32),
        pltpu.VMEM((2 * CHUNK, N_HID), jnp.bfloat16),
        pltpu.VMEM((2 * CHUNK, N_HID), jnp.bfloat16),
        pltpu.VMEM((LANES * LANES,), jnp.float32),
        pltpu.VMEM((PER_W,), jnp.float32),
        pltpu.SemaphoreType.DMA,
        pltpu.SemaphoreType.DMA,
        pltpu.SemaphoreType.DMA,
        pltpu.SemaphoreType.DMA,
    ],
)
def _sc_call(a_hbm, b_hbm, src_hbm, dst_hbm, out_hbm,
             src_v, dst_v, arows_v, brows_v, tile_v, out_v, sem_a0, sem_b0,
             sem_a1, sem_b1):
    _sc_body(a_hbm, b_hbm, src_hbm, dst_hbm, out_hbm,
             src_v, dst_v, arows_v, brows_v, tile_v, out_v, sem_a0, sem_b0,
             sem_a1, sem_b1)


def kernel(emb, edge_index, d, scale):
    ds = (d * scale).reshape(1, N_HID)
    a, b = _prep(emb, ds)
    src = edge_index[0].astype(jnp.int32)
    dst = edge_index[1].astype(jnp.int32)
    z = _sc_call(a, b, src, dst)
    return z.reshape(N_EDGES, 1)


# speedup vs baseline: 7.8016x; 1.0556x over previous
import functools

import jax
import jax.numpy as jnp
from jax import lax
from jax.experimental import pallas as pl
from jax.experimental.pallas import tpu as pltpu
from jax.experimental.pallas import tpu_sc as plsc

N_NODES = 10000
N_HID = 128
N_EDGES = 320000
NW = 32
PER_W = N_EDGES // NW
CHUNK = 80
NCHUNK = PER_W // CHUNK
LANES = 16
NPK = N_HID // (2 * LANES)


def _prep_body(emb_ref, ds_ref, a_ref, b_ref):
    x = emb_ref[...]
    n2 = jnp.sum(x * x, axis=1, keepdims=True)
    inv = 1.0 / jnp.maximum(jnp.sqrt(n2), 1e-12)
    xn = x * inv
    b_ref[...] = xn.astype(jnp.bfloat16)
    a_ref[...] = (xn * ds_ref[...]).astype(jnp.bfloat16)


_ROWS_PER_BLK = 1000


def _prep(emb, ds):
    return pl.pallas_call(
        _prep_body,
        grid=(N_NODES // _ROWS_PER_BLK,),
        in_specs=[
            pl.BlockSpec((_ROWS_PER_BLK, N_HID), lambda i: (i, 0)),
            pl.BlockSpec((1, N_HID), lambda i: (0, 0)),
        ],
        out_specs=[
            pl.BlockSpec((_ROWS_PER_BLK, N_HID), lambda i: (i, 0)),
            pl.BlockSpec((_ROWS_PER_BLK, N_HID), lambda i: (i, 0)),
        ],
        out_shape=[
            jax.ShapeDtypeStruct((N_NODES, N_HID), jnp.bfloat16),
            jax.ShapeDtypeStruct((N_NODES, N_HID), jnp.bfloat16),
        ],
    )(emb, ds)


def _sc_body(a_hbm, b_hbm, src_hbm, dst_hbm, out_hbm,
             src_v, dst_v, arows_v, brows_v, tile_v, out_v, sem_a0, sem_b0,
             sem_a1, sem_b1):
    cid = lax.axis_index("c")
    sid = lax.axis_index("s")
    wid = sid * 2 + cid
    base = wid * PER_W

    pltpu.sync_copy(src_hbm.at[pl.ds(base, PER_W)], src_v)
    pltpu.sync_copy(dst_hbm.at[pl.ds(base, PER_W)], dst_v)

    iota16 = lax.iota(jnp.int32, LANES)
    sems = ((sem_a0, sem_b0), (sem_a1, sem_b1))

    def start(c, b):
        off = c * CHUNK
        pltpu.async_copy(
            a_hbm.at[src_v.at[pl.ds(off, CHUNK)]],
            arows_v.at[pl.ds(b * CHUNK, CHUNK)], sems[b][0])
        pltpu.async_copy(
            b_hbm.at[dst_v.at[pl.ds(off, CHUNK)]],
            brows_v.at[pl.ds(b * CHUNK, CHUNK)], sems[b][1])

    def wait(b):
        pltpu.make_async_copy(
            a_hbm.at[src_v.at[pl.ds(0, CHUNK)]],
            arows_v.at[pl.ds(b * CHUNK, CHUNK)], sems[b][0]).wait()
        pltpu.make_async_copy(
            b_hbm.at[dst_v.at[pl.ds(0, CHUNK)]],
            brows_v.at[pl.ds(b * CHUNK, CHUNK)], sems[b][1]).wait()

    start(0, 0)

    def chunk_body(c, carry):
        off = c * CHUNK
        parity = lax.rem(c, 2)

        @pl.when(c + 1 < NCHUNK)
        def _():
            @pl.when(parity == 0)
            def _():
                start(c + 1, 1)

            @pl.when(parity == 1)
            def _():
                start(c + 1, 0)

        @pl.when(parity == 0)
        def _():
            wait(0)

        @pl.when(parity == 1)
        def _():
            wait(1)

        bofs = parity * CHUNK

        def group_body(g, carry2):
            eoff = g * LANES
            for es in range(LANES):
                e = bofs + eoff + es
                prods = []
                for j in range(NPK):
                    a_pk = arows_v[e, pl.ds(j * 2 * LANES, 2 * LANES)]
                    b_pk = brows_v[e, pl.ds(j * 2 * LANES, 2 * LANES)]
                    prods.append(a_pk * b_pk)
                while len(prods) > 1:
                    prods = [prods[i] + prods[i + 1]
                             for i in range(0, len(prods), 2)]
                u0, u1 = plsc.unpack(
                    prods[0], format=plsc.PackFormat.INTERLEAVED)
                plsc.store_scatter(tile_v, [iota16 * LANES + es], u0 + u1)
            cols = [tile_v[pl.ds(c2 * LANES, LANES)] for c2 in range(LANES)]
            while len(cols) > 1:
                cols = [cols[i] + cols[i + 1] for i in range(0, len(cols), 2)]
            out_v[pl.ds(off + eoff, LANES)] = cols[0]
            return carry2

        lax.fori_loop(0, CHUNK // LANES, group_body, 0)
        return carry

    lax.fori_loop(0, NCHUNK, chunk_body, 0)
    pltpu.sync_copy(out_v, out_hbm.at[pl.ds(base, PER_W)])


@functools.partial(
    pl.kernel,
    out_type=jax.ShapeDtypeStruct((N_EDGES,), jnp.float32),
    mesh=plsc.VectorSubcoreMesh(core_axis_name="c", subcore_axis_name="s"),
    compiler_params=pltpu.CompilerParams(
        needs_layout_passes=False, use_tc_tiling_on_sc=False),
    scratch_types=[
        pltpu.VMEM((PER_W,), jnp.int32),
        pltpu.VMEM((PER_W,), jnp.int32),
        pltpu.VMEM((2 * CHUNK, N_HID), jnp.bfloat16),
        pltpu.VMEM((2 * CHUNK, N_HID), jnp.bfloat16),
        pltpu.VMEM((LANES * LANES,), jnp.float32),
        pltpu.VMEM((PER_W,), jnp.float32),
        pltpu.SemaphoreType.DMA,
        pltpu.SemaphoreType.DMA,
        pltpu.SemaphoreType.DMA,
        pltpu.SemaphoreType.DMA,
    ],
)
def _sc_call(a_hbm, b_hbm, src_hbm, dst_hbm, out_hbm,
             src_v, dst_v, arows_v, brows_v, tile_v, out_v, sem_a0, sem_b0,
             sem_a1, sem_b1):
    _sc_body(a_hbm, b_hbm, src_hbm, dst_hbm, out_hbm,
             src_v, dst_v, arows_v, brows_v, tile_v, out_v, sem_a0, sem_b0,
             sem_a1, sem_b1)


def kernel(emb, edge_index, d, scale):
    ds = (d * scale).reshape(1, N_HID)
    a, b = _prep(emb, ds)
    src = edge_index[0].astype(jnp.int32)
    dst = edge_index[1].astype(jnp.int32)
    z = _sc_call(a, b, src, dst)
    return z.reshape(N_EDGES, 1)
